# baseline (speedup 1.0000x reference)
import functools

import jax
import jax.numpy as jnp
from jax import lax
from jax.experimental import pallas as pl
from jax.experimental.pallas import tpu as pltpu
from jax.experimental.pallas import tpu_sc as plsc

PROMPT_LEN = 16
D = 768
SUF = 110
CTX = 1 + PROMPT_LEN + SUF
HEAD = 24
HSUF = HEAD - 1 - PROMPT_LEN
TAIL = 96
END = CTX - HEAD - TAIL
ROWS = 512
NUM_CORES = 2
NUM_SUBCORES = 16
NW = NUM_CORES * NUM_SUBCORES
RPW = ROWS // NW
CH = 8
NCH = RPW // CH
NBLK = 13
ROWBLK = 64

_mesh = plsc.VectorSubcoreMesh(core_axis_name="c", subcore_axis_name="s")


@functools.partial(
    pl.kernel,
    out_type=(
        jax.ShapeDtypeStruct((ROWS, CTX, D), jnp.float32),
        jax.ShapeDtypeStruct((ROWS, CTX), jnp.int32),
    ),
    mesh=_mesh,
    scratch_types=[
        pltpu.VMEM((RPW,), jnp.int32),
        pltpu.VMEM((CH, PROMPT_LEN, D), jnp.float32),
        pltpu.VMEM((1, CH, D), jnp.float32),
        pltpu.VMEM((RPW, CTX), jnp.int32),
        pltpu.VMEM_SHARED((NUM_SUBCORES, HEAD, D), jnp.float32),
        pltpu.VMEM_SHARED((1, TAIL, D), jnp.float32),
        pltpu.SemaphoreType.DMA,
        pltpu.SemaphoreType.DMA,
        pltpu.SemaphoreType.DMA,
        pltpu.SemaphoreType.DMA,
    ],
)
def _assemble(idx_hbm, prompt_hbm, pre_hbm, suf_hbm, tok_hbm,
              out_emb, out_tok,
              idx_v, core_v, bounce_v, tok_v, head_sh, tail_sh,
              gsem, lsem, osem, wsem):
    cid = lax.axis_index("c")
    sid = lax.axis_index("s")
    wid = sid * NUM_CORES + cid
    base = wid * RPW
    head_v = head_sh.at[pl.ds(sid, 1)]

    pltpu.sync_copy(suf_hbm.at[:, pl.ds(0, CH)], bounce_v)
    pltpu.sync_copy(pre_hbm, head_v.at[:, pl.ds(0, 1)])
    for i in range(HSUF):
        pltpu.sync_copy(bounce_v.at[:, pl.ds(i, 1)],
                        head_v.at[:, pl.ds(1 + PROMPT_LEN + i, 1)])

    @pl.when(sid == 0)
    def _():
        pltpu.sync_copy(bounce_v.at[:, pl.ds(HSUF, 1)],
                        tail_sh.at[:, pl.ds(0, 1)])

    for s in range(1, NBLK):
        @pl.when(sid == s)
        def _():
            pltpu.sync_copy(suf_hbm.at[:, pl.ds(CH * s, CH)], bounce_v)
            for q in range(CH):
                p = CH * s - HSUF + q
                if p < TAIL:
                    pltpu.sync_copy(bounce_v.at[:, pl.ds(q, 1)],
                                    tail_sh.at[:, pl.ds(p, 1)])

    pltpu.sync_copy(idx_hbm.at[pl.ds(base, RPW)], idx_v)
    for k in range(RPW):
        pltpu.sync_copy(tok_hbm, tok_v.at[pl.ds(k, 1)])
    pending = [pltpu.async_copy(tok_v, out_tok.at[pl.ds(base, RPW)], wsem)]

    plsc.subcore_barrier()

    for c in range(NCH):
        r0 = base + c * CH
        pltpu.async_copy(
            prompt_hbm.at[idx_v.at[pl.ds(c * CH, CH)]], core_v, gsem
        ).wait()
        for j in range(CH):
            r = r0 + j
            drops = [
                pltpu.async_copy(core_v.at[pl.ds(j, 1), pl.ds(k, 1)],
                                 head_v.at[:, pl.ds(1 + k, 1)], lsem)
                for k in range(PROMPT_LEN)
            ]
            for d in drops:
                d.wait()
            head_wr = pltpu.async_copy(
                head_v, out_emb.at[pl.ds(r, 1), pl.ds(0, HEAD)], osem)
            pending.append(pltpu.async_copy(
                tail_sh, out_emb.at[pl.ds(r, 1), pl.ds(HEAD, TAIL)], wsem))
            head_wr.wait()

    for p in pending:
        p.wait()


def _end_body(emb_any, suf_ref, out_ref):
    del emb_any
    tailv = suf_ref[0, pl.ds(SUF - END, END), :]
    blk = jnp.concatenate([tailv, jnp.zeros((1, D), jnp.float32)], axis=0)
    out_ref[...] = jnp.broadcast_to(blk[None], (ROWBLK, 8, D))


_end_tile = pl.pallas_call(
    _end_body,
    grid=(ROWS // ROWBLK,),
    in_specs=[
        pl.BlockSpec(memory_space=pl.ANY),
        pl.BlockSpec((1, SUF, D), lambda i: (0, 0, 0)),
    ],
    out_specs=pl.BlockSpec((ROWBLK, 8, D), lambda i: (i, (HEAD + TAIL) // 8, 0)),
    out_shape=jax.ShapeDtypeStruct((ROWS, CTX, D), jnp.float32),
    input_output_aliases={0: 0},
)


def kernel(indices, mini_batch, prompt, embedding_prefix, embedding_suffix,
           tokenized_prompts):
    del mini_batch
    emb, tok = _assemble(indices.reshape(-1), prompt, embedding_prefix,
                         embedding_suffix, tokenized_prompts)
    emb = _end_tile(emb, embedding_suffix)
    return emb, tok


# speedup vs baseline: 1.9301x; 1.9301x over previous
import functools

import jax
import jax.numpy as jnp
from jax import lax
from jax.experimental import pallas as pl
from jax.experimental.pallas import tpu as pltpu
from jax.experimental.pallas import tpu_sc as plsc

PROMPT_LEN = 16
D = 768
SUF = 110
CTX = 1 + PROMPT_LEN + SUF
ROWS = 512
POOL2 = 2048 * PROMPT_LEN
NUM_CORES = 2
NUM_SUBCORES = 16
NW = NUM_CORES * NUM_SUBCORES
RPW = ROWS // NW
NIDX = RPW * PROMPT_LEN
SROWS = 7

_mesh = plsc.VectorSubcoreMesh(core_axis_name="c", subcore_axis_name="s")


@functools.partial(
    pl.kernel,
    out_type=(
        jax.ShapeDtypeStruct((CTX, ROWS, D), jnp.float32),
        jax.ShapeDtypeStruct((ROWS, CTX), jnp.int32),
    ),
    mesh=_mesh,
    scratch_types=[
        pltpu.VMEM((RPW,), jnp.int32),
        pltpu.VMEM((NIDX,), jnp.int32),
        pltpu.VMEM((2, RPW, D), jnp.float32),
        pltpu.VMEM((1, 1, D), jnp.float32),
        pltpu.VMEM((RPW, CTX), jnp.int32),
        pltpu.VMEM_SHARED((SUF, RPW, D), jnp.float32),
        pltpu.VMEM_SHARED((1, RPW, D), jnp.float32),
        pltpu.SemaphoreType.DMA,
        pltpu.SemaphoreType.DMA,
        pltpu.SemaphoreType.DMA,
        pltpu.SemaphoreType.DMA,
    ],
)
def _assemble(idx_hbm, prompt2_hbm, pre_hbm, suf_hbm, tok_hbm,
              out_emb, out_tok,
              idx_v, idx2_v, gbuf, bounce_v, tok_v, suf_rep_sh, pre_rep_sh,
              gsem, lsem, osem, wsem):
    cid = lax.axis_index("c")
    sid = lax.axis_index("s")
    wid = sid * NUM_CORES + cid
    base = wid * RPW

    start = sid * SROWS
    nrows = jnp.minimum(SROWS, SUF - start)

    def _stage_row(q, carry):
        trow = start + q
        pltpu.sync_copy(suf_hbm.at[:, pl.ds(trow, 1)], bounce_v)

        def _rep(rep, c):
            pltpu.sync_copy(bounce_v,
                            suf_rep_sh.at[pl.ds(trow, 1), pl.ds(rep, 1)])
            return c

        return lax.fori_loop(0, RPW, _rep, carry)

    lax.fori_loop(0, nrows, _stage_row, 0)

    @pl.when(sid == 0)
    def _():
        for rep in range(RPW):
            pltpu.sync_copy(pre_hbm, pre_rep_sh.at[:, pl.ds(rep, 1)])

    pltpu.sync_copy(idx_hbm.at[pl.ds(base, RPW)], idx_v)

    def _tok(k, c):
        pltpu.sync_copy(tok_hbm, tok_v.at[pl.ds(k, 1)])
        return c

    lax.fori_loop(0, RPW, _tok, 0)
    tok_wr = pltpu.async_copy(tok_v, out_tok.at[pl.ds(base, RPW)], lsem)

    idx16 = idx_v[...] * PROMPT_LEN
    for k in range(PROMPT_LEN):
        idx2_v[pl.ds(k * RPW, RPW)] = idx16 + k

    plsc.subcore_barrier()

    core_wr = [None, None]
    for k in range(PROMPT_LEN):
        b = k % 2
        if core_wr[b] is not None:
            core_wr[b].wait()
        pltpu.async_copy(
            prompt2_hbm.at[idx2_v.at[pl.ds(k * RPW, RPW)]],
            gbuf.at[b], gsem,
        ).wait()
        core_wr[b] = pltpu.async_copy(
            gbuf.at[pl.ds(b, 1)],
            out_emb.at[pl.ds(1 + k, 1), pl.ds(base, RPW)], osem)

    pltpu.make_async_copy(
        pre_rep_sh, out_emb.at[pl.ds(0, 1), pl.ds(base, RPW)], wsem).start()

    def _bc(t, c):
        pltpu.make_async_copy(
            suf_rep_sh.at[pl.ds(t, 1)],
            out_emb.at[pl.ds(1 + PROMPT_LEN + t, 1), pl.ds(base, RPW)],
            wsem).start()
        return c

    lax.fori_loop(0, SUF, _bc, 0)

    def _drain(t, c):
        pltpu.make_async_copy(
            suf_hbm.at[:, pl.ds(0, RPW)], suf_rep_sh.at[pl.ds(0, 1)],
            wsem).wait()
        return c

    lax.fori_loop(0, SUF + 1, _drain, 0)
    core_wr[0].wait()
    core_wr[1].wait()
    tok_wr.wait()


def kernel(indices, mini_batch, prompt, embedding_prefix, embedding_suffix,
           tokenized_prompts):
    del mini_batch
    emb_t, tok = _assemble(
        indices.reshape(-1),
        prompt.reshape(POOL2, D),
        embedding_prefix, embedding_suffix, tokenized_prompts)
    return jnp.transpose(emb_t, (1, 0, 2)), tok


# speedup vs baseline: 1.9310x; 1.0005x over previous
import functools

import jax
import jax.numpy as jnp
from jax import lax
from jax.experimental import pallas as pl
from jax.experimental.pallas import tpu as pltpu
from jax.experimental.pallas import tpu_sc as plsc

PROMPT_LEN = 16
D = 768
SUF = 110
CTX = 1 + PROMPT_LEN + SUF
ROWS = 512
POOL2 = 2048 * PROMPT_LEN
NUM_CORES = 2
NUM_SUBCORES = 16
NW = NUM_CORES * NUM_SUBCORES
RPW = ROWS // NW
NIDX = RPW * PROMPT_LEN
SROWS = 7

_mesh = plsc.VectorSubcoreMesh(core_axis_name="c", subcore_axis_name="s")


@functools.partial(
    pl.kernel,
    out_type=(
        jax.ShapeDtypeStruct((CTX, ROWS, D), jnp.float32),
        jax.ShapeDtypeStruct((ROWS, CTX), jnp.int32),
    ),
    mesh=_mesh,
    scratch_types=[
        pltpu.VMEM((RPW,), jnp.int32),
        pltpu.VMEM((NIDX,), jnp.int32),
        pltpu.VMEM((2, RPW, D), jnp.float32),
        pltpu.VMEM((SROWS, 1, D), jnp.float32),
        pltpu.VMEM((RPW, CTX), jnp.int32),
        pltpu.VMEM_SHARED((SUF, RPW, D), jnp.float32),
        pltpu.VMEM_SHARED((1, RPW, D), jnp.float32),
        pltpu.SemaphoreType.DMA,
        pltpu.SemaphoreType.DMA,
        pltpu.SemaphoreType.DMA,
        pltpu.SemaphoreType.DMA,
    ],
)
def _assemble(idx_hbm, prompt2_hbm, pre_hbm, suf_hbm, tok_hbm,
              out_emb, out_tok,
              idx_v, idx2_v, gbuf, bounce_v, tok_v, suf_rep_sh, pre_rep_sh,
              gsem, lsem, osem, wsem):
    cid = lax.axis_index("c")
    sid = lax.axis_index("s")
    wid = sid * NUM_CORES + cid
    base = wid * RPW

    start = sid * SROWS
    nrows = jnp.minimum(SROWS, SUF - start)

    def _stage_row(q, carry):
        trow = start + q
        pltpu.sync_copy(suf_hbm.at[:, pl.ds(trow, 1)],
                        bounce_v.at[pl.ds(0, 1)])

        def _rep(rep, c):
            pltpu.sync_copy(bounce_v.at[pl.ds(0, 1)],
                            suf_rep_sh.at[pl.ds(trow, 1), pl.ds(rep, 1)])
            return c

        return lax.fori_loop(0, RPW, _rep, carry)

    lax.fori_loop(0, nrows, _stage_row, 0)

    @pl.when(sid == 0)
    def _():
        for rep in range(RPW):
            pltpu.sync_copy(pre_hbm, pre_rep_sh.at[:, pl.ds(rep, 1)])

    pltpu.sync_copy(idx_hbm.at[pl.ds(base, RPW)], idx_v)

    def _tok(k, c):
        pltpu.sync_copy(tok_hbm, tok_v.at[pl.ds(k, 1)])
        return c

    lax.fori_loop(0, RPW, _tok, 0)
    tok_wr = pltpu.async_copy(tok_v, out_tok.at[pl.ds(base, RPW)], lsem)

    idx16 = idx_v[...] * PROMPT_LEN
    for k in range(PROMPT_LEN):
        idx2_v[pl.ds(k * RPW, RPW)] = idx16 + k

    plsc.subcore_barrier()

    core_wr = [None, None]
    for k in range(PROMPT_LEN):
        b = k % 2
        if core_wr[b] is not None:
            core_wr[b].wait()
        pltpu.async_copy(
            prompt2_hbm.at[idx2_v.at[pl.ds(k * RPW, RPW)]],
            gbuf.at[b], gsem,
        ).wait()
        core_wr[b] = pltpu.async_copy(
            gbuf.at[pl.ds(b, 1)],
            out_emb.at[pl.ds(1 + k, 1), pl.ds(base, RPW)], osem)

    pltpu.make_async_copy(
        pre_rep_sh, out_emb.at[pl.ds(0, 1), pl.ds(base, RPW)], wsem).start()

    def _bc(t, c):
        pltpu.make_async_copy(
            suf_rep_sh.at[pl.ds(t, 1)],
            out_emb.at[pl.ds(1 + PROMPT_LEN + t, 1), pl.ds(base, RPW)],
            wsem).start()
        return c

    lax.fori_loop(0, SUF, _bc, 0)

    def _drain(t, c):
        pltpu.make_async_copy(
            suf_hbm.at[:, pl.ds(0, RPW)], suf_rep_sh.at[pl.ds(0, 1)],
            wsem).wait()
        return c

    lax.fori_loop(0, SUF + 1, _drain, 0)
    core_wr[0].wait()
    core_wr[1].wait()
    tok_wr.wait()


def kernel(indices, mini_batch, prompt, embedding_prefix, embedding_suffix,
           tokenized_prompts):
    del mini_batch
    emb_t, tok = _assemble(
        indices.reshape(-1),
        prompt.reshape(POOL2, D),
        embedding_prefix, embedding_suffix, tokenized_prompts)
    return jnp.transpose(emb_t, (1, 0, 2)), tok


# speedup vs baseline: 2.0540x; 1.0637x over previous
import functools

import jax
import jax.numpy as jnp
from jax import lax
from jax.experimental import pallas as pl
from jax.experimental.pallas import tpu as pltpu
from jax.experimental.pallas import tpu_sc as plsc

PROMPT_LEN = 16
D = 768
SUF = 110
CTX = 1 + PROMPT_LEN + SUF
ROWS = 512
POOL2 = 2048 * PROMPT_LEN
NUM_CORES = 2
NUM_SUBCORES = 16
NW = NUM_CORES * NUM_SUBCORES
RPW = ROWS // NW
NIDX = RPW * PROMPT_LEN
SROWS = 7

_mesh = plsc.VectorSubcoreMesh(core_axis_name="c", subcore_axis_name="s")


@functools.partial(
    pl.kernel,
    out_type=(
        jax.ShapeDtypeStruct((CTX, ROWS, D), jnp.float32),
        jax.ShapeDtypeStruct((ROWS, CTX), jnp.int32),
    ),
    mesh=_mesh,
    scratch_types=[
        pltpu.VMEM((RPW,), jnp.int32),
        pltpu.VMEM((NIDX,), jnp.int32),
        pltpu.VMEM((3, RPW, D), jnp.float32),
        pltpu.VMEM((1, 1, D), jnp.float32),
        pltpu.VMEM((RPW, CTX), jnp.int32),
        pltpu.VMEM_SHARED((SUF, RPW, D), jnp.float32),
        pltpu.VMEM_SHARED((1, RPW, D), jnp.float32),
        pltpu.SemaphoreType.DMA,
        pltpu.SemaphoreType.DMA,
        pltpu.SemaphoreType.DMA,
        pltpu.SemaphoreType.DMA,
    ],
)
def _assemble(idx_hbm, prompt2_hbm, pre_hbm, suf_hbm, tok_hbm,
              out_emb, out_tok,
              idx_v, idx2_v, gbuf, bounce_v, tok_v, suf_rep_sh, pre_rep_sh,
              gsem, lsem, osem, wsem):
    cid = lax.axis_index("c")
    sid = lax.axis_index("s")
    wid = sid * NUM_CORES + cid
    base = wid * RPW

    start = sid * SROWS
    nrows = jnp.minimum(SROWS, SUF - start)

    def _stage_row(q, carry):
        trow = start + q
        pltpu.sync_copy(suf_hbm.at[:, pl.ds(trow, 1)],
                        bounce_v.at[pl.ds(0, 1)])

        def _rep(rep, c):
            pltpu.sync_copy(bounce_v.at[pl.ds(0, 1)],
                            suf_rep_sh.at[pl.ds(trow, 1), pl.ds(rep, 1)])
            return c

        return lax.fori_loop(0, RPW, _rep, carry)

    lax.fori_loop(0, nrows, _stage_row, 0)

    @pl.when(sid == 0)
    def _():
        for rep in range(RPW):
            pltpu.sync_copy(pre_hbm, pre_rep_sh.at[:, pl.ds(rep, 1)])

    pltpu.sync_copy(idx_hbm.at[pl.ds(base, RPW)], idx_v)

    def _tok(k, c):
        pltpu.sync_copy(tok_hbm, tok_v.at[pl.ds(k, 1)])
        return c

    lax.fori_loop(0, RPW, _tok, 0)
    tok_wr = pltpu.async_copy(tok_v, out_tok.at[pl.ds(base, RPW)], lsem)

    idx16 = idx_v[...] * PROMPT_LEN
    for k in range(PROMPT_LEN):
        idx2_v[pl.ds(k * RPW, RPW)] = idx16 + k

    plsc.subcore_barrier()

    pltpu.make_async_copy(
        pre_rep_sh, out_emb.at[pl.ds(0, 1), pl.ds(base, RPW)], wsem).start()

    def _bc(t, c):
        pltpu.make_async_copy(
            suf_rep_sh.at[pl.ds(t, 1)],
            out_emb.at[pl.ds(1 + PROMPT_LEN + t, 1), pl.ds(base, RPW)],
            wsem).start()
        return c

    lax.fori_loop(0, SUF, _bc, 0)

    NB = 3
    gd = [None] * NB
    core_wr = [None] * NB
    for b in range(NB):
        gd[b] = pltpu.async_copy(
            prompt2_hbm.at[idx2_v.at[pl.ds(b * RPW, RPW)]], gbuf.at[b], gsem)
    for k in range(PROMPT_LEN):
        b = k % NB
        gd[b].wait()
        core_wr[b] = pltpu.async_copy(
            gbuf.at[pl.ds(b, 1)],
            out_emb.at[pl.ds(1 + k, 1), pl.ds(base, RPW)], osem)
        if k + NB < PROMPT_LEN:
            core_wr[b].wait()
            gd[b] = pltpu.async_copy(
                prompt2_hbm.at[idx2_v.at[pl.ds((k + NB) * RPW, RPW)]],
                gbuf.at[b], gsem)

    def _drain(t, c):
        pltpu.make_async_copy(
            suf_hbm.at[:, pl.ds(0, RPW)], suf_rep_sh.at[pl.ds(0, 1)],
            wsem).wait()
        return c

    lax.fori_loop(0, SUF + 1, _drain, 0)
    for b in range(NB):
        core_wr[(PROMPT_LEN - NB + b) % NB].wait()
    tok_wr.wait()


def kernel(indices, mini_batch, prompt, embedding_prefix, embedding_suffix,
           tokenized_prompts):
    del mini_batch
    emb_t, tok = _assemble(
        indices.reshape(-1),
        prompt.reshape(POOL2, D),
        embedding_prefix, embedding_suffix, tokenized_prompts)
    return jnp.transpose(emb_t, (1, 0, 2)), tok


# speedup vs baseline: 2.1290x; 1.0365x over previous
import functools

import jax
import jax.numpy as jnp
from jax import lax
from jax.experimental import pallas as pl
from jax.experimental.pallas import tpu as pltpu
from jax.experimental.pallas import tpu_sc as plsc

PROMPT_LEN = 16
D = 768
SUF = 110
CTX = 1 + PROMPT_LEN + SUF
ROWS = 512
POOL2 = 2048 * PROMPT_LEN
NUM_CORES = 2
NUM_SUBCORES = 16
NW = NUM_CORES * NUM_SUBCORES
RPW = ROWS // NW
NIDX = RPW * PROMPT_LEN
SROWS = 7

_mesh = plsc.VectorSubcoreMesh(core_axis_name="c", subcore_axis_name="s")


@functools.partial(
    pl.kernel,
    out_type=(
        jax.ShapeDtypeStruct((CTX, ROWS, D), jnp.float32),
        jax.ShapeDtypeStruct((ROWS, CTX), jnp.int32),
    ),
    mesh=_mesh,
    scratch_types=[
        pltpu.VMEM((RPW,), jnp.int32),
        pltpu.VMEM((NIDX,), jnp.int32),
        pltpu.VMEM((3, RPW, D), jnp.float32),
        pltpu.VMEM((1, 1, D), jnp.float32),
        pltpu.VMEM_SHARED((SUF, RPW, D), jnp.float32),
        pltpu.VMEM_SHARED((1, RPW, D), jnp.float32),
        pltpu.VMEM_SHARED((RPW, CTX), jnp.int32),
        pltpu.SemaphoreType.DMA,
        pltpu.SemaphoreType.DMA,
        pltpu.SemaphoreType.DMA,
        pltpu.SemaphoreType.DMA,
    ],
)
def _assemble(idx_hbm, prompt2_hbm, pre_hbm, suf_hbm, tok_hbm,
              out_emb, out_tok,
              idx_v, idx2_v, gbuf, bounce_v, suf_rep_sh, pre_rep_sh,
              tok_rep_sh, gsem, lsem, osem, wsem):
    cid = lax.axis_index("c")
    sid = lax.axis_index("s")
    wid = sid * NUM_CORES + cid
    base = wid * RPW

    start = sid * SROWS
    nrows = jnp.minimum(SROWS, SUF - start)

    def _stage_row(q, carry):
        trow = start + q
        pltpu.sync_copy(suf_hbm.at[:, pl.ds(trow, 1)],
                        bounce_v.at[pl.ds(0, 1)])

        def _rep(rep, c):
            pltpu.sync_copy(bounce_v.at[pl.ds(0, 1)],
                            suf_rep_sh.at[pl.ds(trow, 1), pl.ds(rep, 1)])
            return c

        return lax.fori_loop(0, RPW, _rep, carry)

    lax.fori_loop(0, nrows, _stage_row, 0)

    @pl.when(sid == NUM_SUBCORES - 1)
    def _():
        for rep in range(RPW):
            pltpu.sync_copy(pre_hbm, pre_rep_sh.at[:, pl.ds(rep, 1)])
            pltpu.sync_copy(tok_hbm, tok_rep_sh.at[pl.ds(rep, 1)])

    pltpu.sync_copy(idx_hbm.at[pl.ds(base, RPW)], idx_v)

    idx16 = idx_v[...] * PROMPT_LEN
    for k in range(PROMPT_LEN):
        idx2_v[pl.ds(k * RPW, RPW)] = idx16 + k

    plsc.subcore_barrier()

    tok_wr = pltpu.async_copy(tok_rep_sh, out_tok.at[pl.ds(base, RPW)], lsem)

    pltpu.make_async_copy(
        pre_rep_sh, out_emb.at[pl.ds(0, 1), pl.ds(base, RPW)], wsem).start()

    def _bc(t, c):
        pltpu.make_async_copy(
            suf_rep_sh.at[pl.ds(t, 1)],
            out_emb.at[pl.ds(1 + PROMPT_LEN + t, 1), pl.ds(base, RPW)],
            wsem).start()
        return c

    lax.fori_loop(0, SUF, _bc, 0)

    NB = 3
    gd = [None] * NB
    core_wr = [None] * NB
    for b in range(NB):
        gd[b] = pltpu.async_copy(
            prompt2_hbm.at[idx2_v.at[pl.ds(b * RPW, RPW)]], gbuf.at[b], gsem)
    for k in range(PROMPT_LEN):
        b = k % NB
        gd[b].wait()
        core_wr[b] = pltpu.async_copy(
            gbuf.at[pl.ds(b, 1)],
            out_emb.at[pl.ds(1 + k, 1), pl.ds(base, RPW)], osem)
        if k + NB < PROMPT_LEN:
            core_wr[b].wait()
            gd[b] = pltpu.async_copy(
                prompt2_hbm.at[idx2_v.at[pl.ds((k + NB) * RPW, RPW)]],
                gbuf.at[b], gsem)

    def _drain(t, c):
        pltpu.make_async_copy(
            suf_hbm.at[:, pl.ds(0, RPW)], suf_rep_sh.at[pl.ds(0, 1)],
            wsem).wait()
        return c

    lax.fori_loop(0, SUF + 1, _drain, 0)
    for b in range(NB):
        core_wr[(PROMPT_LEN - NB + b) % NB].wait()
    tok_wr.wait()


def kernel(indices, mini_batch, prompt, embedding_prefix, embedding_suffix,
           tokenized_prompts):
    del mini_batch
    emb_t, tok = _assemble(
        indices.reshape(-1),
        prompt.reshape(POOL2, D),
        embedding_prefix, embedding_suffix, tokenized_prompts)
    return jnp.transpose(emb_t, (1, 0, 2)), tok


# speedup vs baseline: 2.3381x; 1.0982x over previous
import functools

import jax
import jax.numpy as jnp
from jax import lax
from jax.experimental import pallas as pl
from jax.experimental.pallas import tpu as pltpu
from jax.experimental.pallas import tpu_sc as plsc

PROMPT_LEN = 16
D = 768
SUF = 110
CTX = 1 + PROMPT_LEN + SUF
ROWS = 512
POOL2 = 2048 * PROMPT_LEN
NUM_CORES = 2
NUM_SUBCORES = 16
NW = NUM_CORES * NUM_SUBCORES
RPW = ROWS // NW
REP = 8
NIDX = RPW * PROMPT_LEN
SROWS = 7

_mesh = plsc.VectorSubcoreMesh(core_axis_name="c", subcore_axis_name="s")


@functools.partial(
    pl.kernel,
    out_type=(
        jax.ShapeDtypeStruct((CTX, ROWS, D), jnp.float32),
        jax.ShapeDtypeStruct((ROWS, CTX), jnp.int32),
    ),
    mesh=_mesh,
    scratch_types=[
        pltpu.VMEM((RPW,), jnp.int32),
        pltpu.VMEM((NIDX,), jnp.int32),
        pltpu.VMEM((3, RPW, D), jnp.float32),
        pltpu.VMEM((1, 1, D), jnp.float32),
        pltpu.VMEM_SHARED((SUF, REP, D), jnp.float32),
        pltpu.VMEM_SHARED((1, REP, D), jnp.float32),
        pltpu.VMEM_SHARED((REP, CTX), jnp.int32),
        pltpu.SemaphoreType.DMA,
        pltpu.SemaphoreType.DMA,
        pltpu.SemaphoreType.DMA,
        pltpu.SemaphoreType.DMA,
    ],
)
def _assemble(idx_hbm, prompt2_hbm, pre_hbm, suf_hbm, tok_hbm,
              out_emb, out_tok,
              idx_v, idx2_v, gbuf, bounce_v, suf_rep_sh, pre_rep_sh,
              tok_rep_sh, gsem, lsem, osem, wsem):
    cid = lax.axis_index("c")
    sid = lax.axis_index("s")
    wid = sid * NUM_CORES + cid
    base = wid * RPW

    start = sid * SROWS
    nrows = jnp.minimum(SROWS, SUF - start)

    def _stage_row(q, carry):
        trow = start + q
        pltpu.sync_copy(suf_hbm.at[:, pl.ds(trow, 1)],
                        bounce_v.at[pl.ds(0, 1)])

        def _rep(rep, c):
            pltpu.sync_copy(bounce_v.at[pl.ds(0, 1)],
                            suf_rep_sh.at[pl.ds(trow, 1), pl.ds(rep, 1)])
            return c

        return lax.fori_loop(0, REP, _rep, carry)

    lax.fori_loop(0, nrows, _stage_row, 0)

    @pl.when(sid == NUM_SUBCORES - 1)
    def _():
        for rep in range(REP):
            pltpu.sync_copy(pre_hbm, pre_rep_sh.at[:, pl.ds(rep, 1)])
            pltpu.sync_copy(tok_hbm, tok_rep_sh.at[pl.ds(rep, 1)])

    pltpu.sync_copy(idx_hbm.at[pl.ds(base, RPW)], idx_v)

    idx16 = idx_v[...] * PROMPT_LEN
    for k in range(PROMPT_LEN):
        idx2_v[pl.ds(k * RPW, RPW)] = idx16 + k

    plsc.subcore_barrier()

    tok_wr = [
        pltpu.async_copy(tok_rep_sh, out_tok.at[pl.ds(base + h * REP, REP)],
                         lsem)
        for h in range(RPW // REP)
    ]

    for h in range(RPW // REP):
        pltpu.make_async_copy(
            pre_rep_sh, out_emb.at[pl.ds(0, 1), pl.ds(base + h * REP, REP)],
            wsem).start()

    def _bc(t, c):
        for h in range(RPW // REP):
            pltpu.make_async_copy(
                suf_rep_sh.at[pl.ds(t, 1)],
                out_emb.at[pl.ds(1 + PROMPT_LEN + t, 1),
                           pl.ds(base + h * REP, REP)],
                wsem).start()
        return c

    lax.fori_loop(0, SUF, _bc, 0)

    NB = 3
    gd = [None] * NB
    core_wr = [None] * NB
    for b in range(NB):
        gd[b] = pltpu.async_copy(
            prompt2_hbm.at[idx2_v.at[pl.ds(b * RPW, RPW)]], gbuf.at[b], gsem)
    for k in range(PROMPT_LEN):
        b = k % NB
        gd[b].wait()
        core_wr[b] = pltpu.async_copy(
            gbuf.at[pl.ds(b, 1)],
            out_emb.at[pl.ds(1 + k, 1), pl.ds(base, RPW)], osem)
        if k + NB < PROMPT_LEN:
            core_wr[b].wait()
            gd[b] = pltpu.async_copy(
                prompt2_hbm.at[idx2_v.at[pl.ds((k + NB) * RPW, RPW)]],
                gbuf.at[b], gsem)

    def _drain(t, c):
        pltpu.make_async_copy(
            suf_hbm.at[:, pl.ds(0, REP)], suf_rep_sh.at[pl.ds(0, 1)],
            wsem).wait()
        return c

    lax.fori_loop(0, (SUF + 1) * (RPW // REP), _drain, 0)
    for b in range(NB):
        core_wr[(PROMPT_LEN - NB + b) % NB].wait()
    for w in tok_wr:
        w.wait()


def kernel(indices, mini_batch, prompt, embedding_prefix, embedding_suffix,
           tokenized_prompts):
    del mini_batch
    emb_t, tok = _assemble(
        indices.reshape(-1),
        prompt.reshape(POOL2, D),
        embedding_prefix, embedding_suffix, tokenized_prompts)
    return jnp.transpose(emb_t, (1, 0, 2)), tok
